# baseline (speedup 1.0000x reference)
import functools

import jax
import jax.numpy as jnp
import numpy as np
from jax import lax
from jax.experimental import pallas as pl
from jax.experimental.pallas import tpu as pltpu
from jax.experimental.pallas import tpu_sc as plsc

_BINS = (-1.1503, -0.6745, -0.3186, 0.0, 0.3186, 0.6745, 1.1503)

_N = 32768
_NW = 32
_CHUNK = _N // _NW
_L = 16
_G = _CHUNK // 128

_R = np.arange(_N, dtype=np.int64)
_IDX = ((_R >> 3) * 512 + (_R & 7) * 8).astype(np.int32).reshape(_NW, _G, 128)


def _bucketize_body(x_hbm, idx_hbm, out_hbm, idx_v, rows_v, r_v, sem):
    c = lax.axis_index("c")
    s = lax.axis_index("s")
    wid = s * 2 + c
    base = wid * _CHUNK

    lane = lax.iota(jnp.int32, _L)
    zero = jnp.zeros((_L,), jnp.int32)

    pltpu.sync_copy(idx_hbm.at[wid], idx_v)
    cps = [
        pltpu.async_copy(x_hbm.at[idx_v.at[g]], rows_v.at[g], sem)
        for g in range(_G)
    ]

    for g in range(_G):
        cps[g].wait()

        def group(h, carry):
            y = plsc.load_gather(rows_v, [g + zero, h * _L + lane, zero])
            r = jnp.zeros((_L,), jnp.int32)
            for b in _BINS:
                r = r + (y > jnp.float32(b)).astype(jnp.int32)
            r_v[pl.ds((g * 8 + h) * _L, _L)] = r
            return carry

        lax.fori_loop(0, 128 // _L, group, 0)

    pltpu.sync_copy(r_v, out_hbm.at[pl.ds(base, _CHUNK)])


def kernel(x):
    xf = (
        x.reshape(8192, 8, 8, 128)
        .transpose(0, 2, 1, 3)
        .reshape(2 * _N * 64, 16)
    )
    mesh = plsc.VectorSubcoreMesh(core_axis_name="c", subcore_axis_name="s")
    run = functools.partial(
        pl.kernel,
        mesh=mesh,
        out_type=jax.ShapeDtypeStruct((_N,), jnp.int32),
        scratch_types=[
            pltpu.VMEM((_G, 128), jnp.int32),
            pltpu.VMEM((_G, 128, 16), jnp.float32),
            pltpu.VMEM((_CHUNK,), jnp.int32),
            pltpu.SemaphoreType.DMA,
        ],
        compiler_params=pltpu.CompilerParams(
            use_tc_tiling_on_sc=False,
            needs_layout_passes=False,
        ),
    )(_bucketize_body)
    return run(xf, jnp.asarray(_IDX))


# speedup vs baseline: 1.0162x; 1.0162x over previous
import functools

import jax
import jax.numpy as jnp
import numpy as np
from jax import lax
from jax.experimental import pallas as pl
from jax.experimental.pallas import tpu as pltpu
from jax.experimental.pallas import tpu_sc as plsc

_BINS = (-1.1503, -0.6745, -0.3186, 0.0, 0.3186, 0.6745, 1.1503)

_N = 32768
_NW = 32
_CHUNK = _N // _NW
_L = 16
_G = _CHUNK // 128

def _bucketize_body(x_hbm, out_hbm, idx_v, rows_v, r_v, sem):
    c = lax.axis_index("c")
    s = lax.axis_index("s")
    wid = s * 2 + c
    base = wid * _CHUNK

    lane = lax.iota(jnp.int32, _L)
    zero = jnp.zeros((_L,), jnp.int32)

    cps = []
    for g in range(_G):
        def idx_group(h, carry, g=g):
            r = base + (g * 8 + h) * _L + lane
            idx_v[g, pl.ds(h * _L, _L)] = ((r >> 3) << 9) + ((r & 7) << 3)
            return carry

        lax.fori_loop(0, 128 // _L, idx_group, 0)
        cps.append(pltpu.async_copy(x_hbm.at[idx_v.at[g]], rows_v.at[g], sem))

    for g in range(_G):
        cps[g].wait()

        def group(h, carry):
            y = plsc.load_gather(rows_v, [g + zero, h * _L + lane, zero])
            r = jnp.zeros((_L,), jnp.int32)
            for b in _BINS:
                r = r + (y > jnp.float32(b)).astype(jnp.int32)
            r_v[pl.ds((g * 8 + h) * _L, _L)] = r
            return carry

        lax.fori_loop(0, 128 // _L, group, 0)

    pltpu.sync_copy(r_v, out_hbm.at[pl.ds(base, _CHUNK)])


def kernel(x):
    xf = (
        x.reshape(8192, 8, 8, 128)
        .transpose(0, 2, 1, 3)
        .reshape(2 * _N * 64, 16)
    )
    mesh = plsc.VectorSubcoreMesh(core_axis_name="c", subcore_axis_name="s")
    run = functools.partial(
        pl.kernel,
        mesh=mesh,
        out_type=jax.ShapeDtypeStruct((_N,), jnp.int32),
        scratch_types=[
            pltpu.VMEM((_G, 128), jnp.int32),
            pltpu.VMEM((_G, 128, 16), jnp.float32),
            pltpu.VMEM((_CHUNK,), jnp.int32),
            pltpu.SemaphoreType.DMA,
        ],
        compiler_params=pltpu.CompilerParams(
            use_tc_tiling_on_sc=False,
            needs_layout_passes=False,
        ),
    )(_bucketize_body)
    return run(xf)


# speedup vs baseline: 1.0357x; 1.0191x over previous
import functools

import jax
import jax.numpy as jnp
import numpy as np
from jax import lax
from jax.experimental import pallas as pl
from jax.experimental.pallas import tpu as pltpu
from jax.experimental.pallas import tpu_sc as plsc

_BINS = (-1.1503, -0.6745, -0.3186, 0.0, 0.3186, 0.6745, 1.1503)

_N = 32768
_NW = 32
_CHUNK = _N // _NW
_L = 16
_G = _CHUNK // 128

def _bucketize_body(x_hbm, out_hbm, idx_v, rows_v, r_v, sem):
    c = lax.axis_index("c")
    s = lax.axis_index("s")
    wid = s * 2 + c
    base = wid * _CHUNK

    lane = lax.iota(jnp.int32, _L)
    zero = jnp.zeros((_L,), jnp.int32)

    def idx_group(g, carry):
        r = base + g * _L + lane
        idx_v[g // 8, pl.ds((g % 8) * _L, _L)] = ((r >> 3) << 9) + ((r & 7) << 3)
        return carry

    lax.fori_loop(0, _CHUNK // _L, idx_group, 0, unroll=4)

    cps = [
        pltpu.async_copy(x_hbm.at[idx_v.at[g]], rows_v.at[g], sem)
        for g in range(_G)
    ]
    for cp in cps:
        cp.wait()

    def group(g, carry):
        y = plsc.load_gather(rows_v, [g // 8 + zero, (g % 8) * _L + lane, zero])
        r = jnp.zeros((_L,), jnp.int32)
        for b in _BINS:
            r = r + (y > jnp.float32(b)).astype(jnp.int32)
        r_v[pl.ds(g * _L, _L)] = r
        return carry

    lax.fori_loop(0, _CHUNK // _L, group, 0, unroll=4)

    pltpu.sync_copy(r_v, out_hbm.at[pl.ds(base, _CHUNK)])


def kernel(x):
    xf = (
        x.reshape(8192, 8, 8, 128)
        .transpose(0, 2, 1, 3)
        .reshape(2 * _N * 64, 16)
    )
    mesh = plsc.VectorSubcoreMesh(core_axis_name="c", subcore_axis_name="s")
    run = functools.partial(
        pl.kernel,
        mesh=mesh,
        out_type=jax.ShapeDtypeStruct((_N,), jnp.int32),
        scratch_types=[
            pltpu.VMEM((_G, 128), jnp.int32),
            pltpu.VMEM((_G, 128, 16), jnp.float32),
            pltpu.VMEM((_CHUNK,), jnp.int32),
            pltpu.SemaphoreType.DMA,
        ],
        compiler_params=pltpu.CompilerParams(
            use_tc_tiling_on_sc=False,
            needs_layout_passes=False,
        ),
    )(_bucketize_body)
    return run(xf)


# speedup vs baseline: 1.0379x; 1.0022x over previous
import functools

import jax
import jax.numpy as jnp
from jax import lax
from jax.experimental import pallas as pl
from jax.experimental.pallas import tpu as pltpu
from jax.experimental.pallas import tpu_sc as plsc

_BINS = (-1.1503, -0.6745, -0.3186, 0.0, 0.3186, 0.6745, 1.1503)

_N = 32768
_NW = 32
_CHUNK = _N // _NW
_L = 16
_G = _CHUNK // 128


def _bucketize_body(x_hbm, out_hbm, idx_v, rows_v, r_v, sem):
    c = lax.axis_index("c")
    s = lax.axis_index("s")
    wid = s * 2 + c
    base = wid * _CHUNK

    lane = lax.iota(jnp.int32, _L)
    zero = jnp.zeros((_L,), jnp.int32)

    def idx_group(g, carry):
        r = base + g * _L + lane
        idx_v[g // 8, pl.ds((g % 8) * _L, _L)] = ((r >> 3) << 9) + ((r & 7) << 3)
        return carry

    lax.fori_loop(0, _CHUNK // _L, idx_group, 0, unroll=4)

    cps = [
        pltpu.async_copy(x_hbm.at[idx_v.at[g]], rows_v.at[g], sem)
        for g in range(_G)
    ]
    for cp in cps:
        cp.wait()

    def group(g, carry):
        y = plsc.load_gather(rows_v, [g // 8 + zero, (g % 8) * _L + lane, zero])
        r = jnp.zeros((_L,), jnp.int32)
        for b in _BINS:
            r = r + (y > jnp.float32(b)).astype(jnp.int32)
        r_v[pl.ds(g * _L, _L)] = r
        return carry

    lax.fori_loop(0, _CHUNK // _L, group, 0, unroll=4)

    pltpu.sync_copy(r_v, out_hbm.at[pl.ds(base, _CHUNK)])


def kernel(x):
    xf = (
        x.reshape(8192, 8, 8, 128)
        .transpose(0, 2, 1, 3)
        .reshape(2 * _N * 64, 16)
    )
    mesh = plsc.VectorSubcoreMesh(core_axis_name="c", subcore_axis_name="s")
    run = functools.partial(
        pl.kernel,
        mesh=mesh,
        out_type=jax.ShapeDtypeStruct((_N,), jnp.int32),
        scratch_types=[
            pltpu.VMEM((_G, 128), jnp.int32),
            pltpu.VMEM((_G, 128, 16), jnp.float32),
            pltpu.VMEM((_CHUNK,), jnp.int32),
            pltpu.SemaphoreType.DMA,
        ],
        compiler_params=pltpu.CompilerParams(
            use_tc_tiling_on_sc=False,
            needs_layout_passes=False,
        ),
    )(_bucketize_body)
    return run(xf)
